# Initial kernel scaffold; baseline (speedup 1.0000x reference)
import jax
import jax.numpy as jnp
from jax.experimental import pallas as pl


def kernel(node_features, deg_slice, membership, gc0_W, gc0_b, gc1_W, gc1_b, bn0_g, bn0_b, bn1_g, bn1_b, dense0_W, dense0_b, bn2_g, bn2_b, dense1_W, dense1_b, deg_adj_1, deg_adj_2, deg_adj_3, deg_adj_4, deg_adj_5, deg_adj_6, deg_adj_7, deg_adj_8, deg_adj_9, deg_adj_10):
    raise NotImplementedError("write your pallas kernel here")


# speedup vs baseline: 2.7691x; 2.7691x over previous
import functools

import jax
import jax.numpy as jnp
from jax import lax
from jax.experimental import pallas as pl
from jax.experimental.pallas import tpu as pltpu
from jax.experimental.pallas import tpu_sc as plsc

N = 100000
PER = 10000
MAXD = 10
NG = 128
C = 80
NCH = 125
NW = 32

_ROT = [0] + [(29 - 3 * (10 - d)) % 32 for d in range(1, 11)]


def _mesh():
    return plsc.VectorSubcoreMesh(core_axis_name="c", subcore_axis_name="s")


def _wid():
    return lax.axis_index("s") * 2 + lax.axis_index("c")


def _sc_gather_sum(src, adj2, feat):
    nls = feat // 16

    @functools.partial(
        pl.kernel,
        out_type=jax.ShapeDtypeStruct((N, feat), jnp.float32),
        mesh=_mesh(),
        scratch_types=[
            pltpu.VMEM((16, C), jnp.int32),
            pltpu.VMEM((MAXD * C, feat), jnp.float32),
            pltpu.VMEM((C, feat), jnp.float32),
        ],
    )
    def k(src_h, adj_h, out_h, idx_v, buf, outb):
        w = _wid()
        for d in range(1, MAXD + 1):
            start = lax.rem(w + _ROT[d], 32)
            nch = jnp.where(start < 29, 4, 3).astype(jnp.int32)

            @pl.loop(0, 4)
            def _(kk, d=d, start=start, nch=nch):
                @pl.when(kk < nch)
                def _(kk=kk, d=d, start=start):
                    c = start + 32 * kk
                    t = (d - 1) * NCH + c
                    pltpu.sync_copy(adj_h.at[t], idx_v)
                    for j in range(d):
                        pltpu.sync_copy(src_h.at[idx_v.at[j]],
                                        buf.at[pl.ds(j * C, C)])

                    @pl.loop(0, C)
                    def _(i, d=d):
                        for cc in range(nls):
                            sl = pl.ds(cc * 16, 16)
                            acc = buf[i, sl]
                            for j in range(1, d):
                                acc = acc + buf[j * C + i, sl]
                            outb[i, sl] = acc

                    node0 = (d - 1) * PER + c * C
                    pltpu.sync_copy(outb, out_h.at[pl.ds(node0, C)])

    return k(src, adj2)


def _sc_gather_max(src, adj2, so, feat):
    nls = feat // 16

    @functools.partial(
        pl.kernel,
        out_type=jax.ShapeDtypeStruct((N, feat), jnp.float32),
        mesh=_mesh(),
        scratch_types=[
            pltpu.VMEM((16, C), jnp.int32),
            pltpu.VMEM((MAXD * C, feat), jnp.float32),
            pltpu.VMEM((C, feat), jnp.float32),
            pltpu.VMEM((C, feat), jnp.float32),
            pltpu.VMEM((2, feat), jnp.float32),
        ],
    )
    def k(src_h, adj_h, so_h, out_h, idx_v, buf, selfb, outb, so_v):
        w = _wid()
        pltpu.sync_copy(so_h, so_v)
        scs = [so_v[0, pl.ds(cc * 16, 16)] for cc in range(nls)]
        ofs = [so_v[1, pl.ds(cc * 16, 16)] for cc in range(nls)]
        for d in range(1, MAXD + 1):
            start = lax.rem(w + _ROT[d], 32)
            nch = jnp.where(start < 29, 4, 3).astype(jnp.int32)

            @pl.loop(0, 4)
            def _(kk, d=d, start=start, nch=nch):
                @pl.when(kk < nch)
                def _(kk=kk, d=d, start=start):
                    c = start + 32 * kk
                    t = (d - 1) * NCH + c
                    node0 = (d - 1) * PER + c * C
                    pltpu.sync_copy(adj_h.at[t], idx_v)
                    for j in range(d):
                        pltpu.sync_copy(src_h.at[idx_v.at[j]],
                                        buf.at[pl.ds(j * C, C)])
                    pltpu.sync_copy(src_h.at[pl.ds(node0, C)], selfb)

                    @pl.loop(0, C)
                    def _(i, d=d):
                        for cc in range(nls):
                            sl = pl.ds(cc * 16, 16)
                            acc = selfb[i, sl] * scs[cc] + ofs[cc]
                            for j in range(d):
                                r = buf[j * C + i, sl] * scs[cc] + ofs[cc]
                                acc = jnp.maximum(acc, r)
                            outb[i, sl] = acc

                    pltpu.sync_copy(outb, out_h.at[pl.ds(node0, C)])

    return k(src, adj2, so)


def _sc_segment(h2, mem, so):
    nls = 8

    @functools.partial(
        pl.kernel,
        out_type=(
            jax.ShapeDtypeStruct((2, NG, 128), jnp.float32),
            jax.ShapeDtypeStruct((2, NG, 128), jnp.float32),
            jax.ShapeDtypeStruct((2, NG, 16), jnp.float32),
        ),
        mesh=_mesh(),
        scratch_types=[
            pltpu.VMEM((64,), jnp.int32),
            pltpu.VMEM((64, 128), jnp.float32),
            pltpu.VMEM((NG, 128), jnp.float32),
            pltpu.VMEM((NG, 128), jnp.float32),
            pltpu.VMEM((NG, 16), jnp.float32),
            pltpu.VMEM((2, 128), jnp.float32),
            pltpu.VMEM((8, 128), jnp.float32),
            pltpu.VMEM((8, 16), jnp.float32),
            pltpu.VMEM_SHARED((16, NG, 128), jnp.float32),
            pltpu.VMEM_SHARED((16, NG, 128), jnp.float32),
            pltpu.VMEM_SHARED((16, NG, 16), jnp.float32),
        ],
    )
    def k(h2_h, mem_h, so_h, sum_o, max_o, cnt_o,
          memb, hbuf, sum_t, max_t, cnt_t, so_v, redb, cntr,
          sum_s, max_s, cnt_s):
        cid = lax.axis_index("c")
        sid = lax.axis_index("s")
        w = sid * 2 + cid
        pltpu.sync_copy(so_h, so_v)
        scs = [so_v[0, pl.ds(cc * 16, 16)] for cc in range(nls)]
        ofs = [so_v[1, pl.ds(cc * 16, 16)] for cc in range(nls)]
        zero = jnp.zeros((16,), jnp.float32)
        neg = jnp.full((16,), -3.4e38, jnp.float32)
        one0 = jnp.where(lax.iota(jnp.int32, 16) == 0, 1.0, 0.0
                         ).astype(jnp.float32)

        @pl.loop(0, NG)
        def _(r):
            for cc in range(nls):
                sl = pl.ds(cc * 16, 16)
                sum_t[r, sl] = zero
                max_t[r, sl] = neg
            cnt_t[r, :] = zero

        def group_body(i0):
            mvec = memb[pl.ds(i0, 16)]
            for ln in range(16):
                m = mvec[ln]
                i = i0 + ln
                for cc in range(nls):
                    sl = pl.ds(cc * 16, 16)
                    r = hbuf[i, sl] * scs[cc] + ofs[cc]
                    sum_t[m, sl] = sum_t[m, sl] + r
                    max_t[m, sl] = jnp.maximum(max_t[m, sl], r)
                cnt_t[m, :] = cnt_t[m, :] + one0

        nc = jnp.where(w < 26, 49, 48).astype(jnp.int32)

        @pl.loop(0, 49)
        def _(kk):
            @pl.when(kk < nc)
            def _(kk=kk):
                row0 = (w + 32 * kk) * 64
                pltpu.sync_copy(mem_h.at[pl.ds(row0, 64)], memb)
                pltpu.sync_copy(h2_h.at[pl.ds(row0, 64)], hbuf)

                @pl.loop(0, 64, step=16)
                def _(i0):
                    group_body(i0)

        @pl.when(w == 0)
        def _():
            pltpu.sync_copy(mem_h.at[pl.ds(99968, 32)], memb.at[pl.ds(0, 32)])
            pltpu.sync_copy(h2_h.at[pl.ds(99968, 32)], hbuf.at[pl.ds(0, 32)])

            @pl.loop(0, 32, step=16)
            def _(i0):
                group_body(i0)

        pltpu.sync_copy(sum_t, sum_s.at[sid])
        pltpu.sync_copy(max_t, max_s.at[sid])
        pltpu.sync_copy(cnt_t, cnt_s.at[sid])
        plsc.subcore_barrier()

        r0 = sid * 8
        for rr in range(8):
            for cc in range(nls):
                sum_t[rr, pl.ds(cc * 16, 16)] = zero
                max_t[rr, pl.ds(cc * 16, 16)] = neg
            cnt_t[rr, :] = zero

        @pl.loop(0, 16)
        def _(t):
            pltpu.sync_copy(sum_s.at[t, pl.ds(r0, 8), :], redb)
            for rr in range(8):
                for cc in range(nls):
                    sl = pl.ds(cc * 16, 16)
                    sum_t[rr, sl] = sum_t[rr, sl] + redb[rr, sl]
            pltpu.sync_copy(max_s.at[t, pl.ds(r0, 8), :], redb)
            for rr in range(8):
                for cc in range(nls):
                    sl = pl.ds(cc * 16, 16)
                    max_t[rr, sl] = jnp.maximum(max_t[rr, sl], redb[rr, sl])
            pltpu.sync_copy(cnt_s.at[t, pl.ds(r0, 8), :], cntr)
            for rr in range(8):
                cnt_t[rr, :] = cnt_t[rr, :] + cntr[rr, :]

        pltpu.sync_copy(sum_t.at[pl.ds(0, 8)], sum_o.at[cid, pl.ds(r0, 8), :])
        pltpu.sync_copy(max_t.at[pl.ds(0, 8)], max_o.at[cid, pl.ds(r0, 8), :])
        pltpu.sync_copy(cnt_t.at[pl.ds(0, 8)], cnt_o.at[cid, pl.ds(r0, 8), :])

    return k(h2, mem, so)


def _tc_gconv(neigh, selfx, wn, ws, b, g, bb, f_in):
    def body(n_ref, s_ref, wn_ref, ws_ref, b_ref, g_ref, bb_ref,
             h_ref, so_ref, acc_ref):
        d = pl.program_id(0)
        i = pl.program_id(1)

        @pl.when((d == 0) & (i == 0))
        def _():
            acc_ref[...] = jnp.zeros_like(acc_ref)

        h = jnp.dot(n_ref[...], wn_ref[0], preferred_element_type=jnp.float32)
        h = h + jnp.dot(s_ref[...], ws_ref[0],
                        preferred_element_type=jnp.float32)
        h = jnp.maximum(h + b_ref[0, 0], 0.0)
        h_ref[...] = jnp.concatenate(
            [h, jnp.zeros((1000, 64), jnp.float32)], axis=1)
        acc_ref[0, :] = acc_ref[0, :] + jnp.sum(h, axis=0)
        acc_ref[1, :] = acc_ref[1, :] + jnp.sum(h * h, axis=0)

        @pl.when((d == 9) & (i == 9))
        def _():
            mean = acc_ref[0, :] / N
            var = acc_ref[1, :] / N - mean * mean
            scale = g_ref[0] * lax.rsqrt(var + 1e-5)
            pad = jnp.zeros((64,), jnp.float32)
            so_ref[0, :] = jnp.concatenate([scale, pad])
            so_ref[1, :] = jnp.concatenate([bb_ref[0] - mean * scale, pad])

    return pl.pallas_call(
        body,
        grid=(10, 10),
        in_specs=[
            pl.BlockSpec((1000, f_in), lambda d, i: (d * 10 + i, 0)),
            pl.BlockSpec((1000, f_in), lambda d, i: (d * 10 + i, 0)),
            pl.BlockSpec((1, f_in, 64), lambda d, i: (d, 0, 0)),
            pl.BlockSpec((1, f_in, 64), lambda d, i: (d, 0, 0)),
            pl.BlockSpec((1, 1, 64), lambda d, i: (d, 0, 0)),
            pl.BlockSpec((1, 64), lambda d, i: (0, 0)),
            pl.BlockSpec((1, 64), lambda d, i: (0, 0)),
        ],
        out_specs=[
            pl.BlockSpec((1000, 128), lambda d, i: (d * 10 + i, 0)),
            pl.BlockSpec((2, 128), lambda d, i: (0, 0)),
        ],
        out_shape=[
            jax.ShapeDtypeStruct((N, 128), jnp.float32),
            jax.ShapeDtypeStruct((2, 128), jnp.float32),
        ],
        scratch_shapes=[pltpu.VMEM((2, 64), jnp.float32)],
    )(neigh, selfx, wn, ws, b.reshape(10, 1, 64), g.reshape(1, -1),
      bb.reshape(1, -1))


def _tc_dense(x2, w, b, g, bb):
    def body(x_ref, w_ref, b_ref, g_ref, bb_ref, h_ref, so_ref, acc_ref):
        i = pl.program_id(0)

        @pl.when(i == 0)
        def _():
            acc_ref[...] = jnp.zeros_like(acc_ref)

        h = jnp.dot(x_ref[...], w_ref[...], preferred_element_type=jnp.float32)
        h = jnp.maximum(h + b_ref[0], 0.0)
        h_ref[...] = h
        acc_ref[0, :] = acc_ref[0, :] + jnp.sum(h, axis=0)
        acc_ref[1, :] = acc_ref[1, :] + jnp.sum(h * h, axis=0)

        @pl.when(i == 99)
        def _():
            mean = acc_ref[0, :] / N
            var = acc_ref[1, :] / N - mean * mean
            scale = g_ref[0] * lax.rsqrt(var + 1e-5)
            so_ref[0, :] = scale
            so_ref[1, :] = bb_ref[0] - mean * scale

    return pl.pallas_call(
        body,
        grid=(100,),
        in_specs=[
            pl.BlockSpec((1000, 128), lambda i: (i, 0)),
            pl.BlockSpec((128, 128), lambda i: (0, 0)),
            pl.BlockSpec((1, 128), lambda i: (0, 0)),
            pl.BlockSpec((1, 128), lambda i: (0, 0)),
            pl.BlockSpec((1, 128), lambda i: (0, 0)),
        ],
        out_specs=[
            pl.BlockSpec((1000, 128), lambda i: (i, 0)),
            pl.BlockSpec((2, 128), lambda i: (0, 0)),
        ],
        out_shape=[
            jax.ShapeDtypeStruct((N, 128), jnp.float32),
            jax.ShapeDtypeStruct((2, 128), jnp.float32),
        ],
        scratch_shapes=[pltpu.VMEM((2, 128), jnp.float32)],
    )(x2, w, b.reshape(1, -1), g.reshape(1, -1), bb.reshape(1, -1))


def _tc_final(ssum, smax, scnt, w1, b1):
    def body(s_ref, m_ref, c_ref, w_ref, b_ref, o_ref):
        s = s_ref[0] + s_ref[1]
        m = jnp.maximum(m_ref[0], m_ref[1])
        cnt = c_ref[0, :, 0:1] + c_ref[1, :, 0:1]
        gg = jnp.concatenate([s / cnt, m], axis=1)
        o_ref[...] = jnp.dot(gg, w_ref[...],
                             preferred_element_type=jnp.float32) + b_ref[0]

    return pl.pallas_call(
        body,
        out_shape=jax.ShapeDtypeStruct((NG, 2), jnp.float32),
    )(ssum, smax, scnt, w1, b1.reshape(1, -1))


def kernel(node_features, deg_slice, membership, gc0_W, gc0_b, gc1_W, gc1_b,
           bn0_g, bn0_b, bn1_g, bn1_b, dense0_W, dense0_b, bn2_g, bn2_b,
           dense1_W, dense1_b, deg_adj_1, deg_adj_2, deg_adj_3, deg_adj_4,
           deg_adj_5, deg_adj_6, deg_adj_7, deg_adj_8, deg_adj_9, deg_adj_10):
    adjs = [deg_adj_1, deg_adj_2, deg_adj_3, deg_adj_4, deg_adj_5,
            deg_adj_6, deg_adj_7, deg_adj_8, deg_adj_9, deg_adj_10]
    parts = []
    for d, a in enumerate(adjs, 1):
        a32 = a.astype(jnp.int32)
        p = a32.reshape(NCH, C, d).transpose(0, 2, 1)
        p = jnp.pad(p, ((0, 0), (0, 16 - d), (0, 0)))
        parts.append(p)
    adj2 = jnp.concatenate(parts, axis=0)
    mem32 = membership.astype(jnp.int32)

    wn0, ws0 = gc0_W[1::2], gc0_W[2::2]
    b0 = gc0_b[1::2] + gc0_b[2::2]
    pad_w = ((0, 0), (0, 64), (0, 0))
    wn1 = jnp.pad(gc1_W[1::2], pad_w)
    ws1 = jnp.pad(gc1_W[2::2], pad_w)
    b1 = gc1_b[1::2] + gc1_b[2::2]
    d0w = jnp.pad(dense0_W, ((0, 64), (0, 0)))

    neigh0 = _sc_gather_sum(node_features, adj2, 128)
    h0, so0 = _tc_gconv(neigh0, node_features, wn0, ws0, b0, bn0_g, bn0_b, 128)
    x1 = _sc_gather_max(h0, adj2, so0, 128)
    neigh1 = _sc_gather_sum(x1, adj2, 128)
    h1, so1 = _tc_gconv(neigh1, x1, wn1, ws1, b1, bn1_g, bn1_b, 128)
    x2 = _sc_gather_max(h1, adj2, so1, 128)
    h2, so2 = _tc_dense(x2, d0w, dense0_b, bn2_g, bn2_b)
    ssum, smax, scnt = _sc_segment(h2, mem32, so2)
    return _tc_final(ssum, smax, scnt, dense1_W, dense1_b)


# speedup vs baseline: 3.4119x; 1.2321x over previous
import functools

import jax
import jax.numpy as jnp
from jax import lax
from jax.experimental import pallas as pl
from jax.experimental.pallas import tpu as pltpu
from jax.experimental.pallas import tpu_sc as plsc

N = 100000
PER = 10000
MAXD = 10
NG = 128
C = 80
NCH = 125
NW = 32

_ROT = [0] + [(29 - 3 * (10 - d)) % 32 for d in range(1, 11)]


def _mesh():
    return plsc.VectorSubcoreMesh(core_axis_name="c", subcore_axis_name="s")


def _wid():
    return lax.axis_index("s") * 2 + lax.axis_index("c")


def _prefetch_idx(adj_h, idx4, sidx, t0, nch):
    cps = [pltpu.async_copy(adj_h.at[t0 + 32 * kk], idx4.at[kk], sidx)
           for kk in range(3)]

    @pl.when(nch == 4)
    def _():
        pltpu.sync_copy(adj_h.at[t0 + 32 * 3], idx4.at[3])

    for cp in cps:
        cp.wait()


def _sc_gather_sum(src, adj2, feat):
    nls = feat // 16

    @functools.partial(
        pl.kernel,
        out_type=jax.ShapeDtypeStruct((N, feat), jnp.float32),
        mesh=_mesh(),
        scratch_types=[
            pltpu.VMEM((4, 16, C), jnp.int32),
            pltpu.VMEM((C, feat), jnp.float32),
            pltpu.VMEM((C, feat), jnp.float32),
            pltpu.VMEM((C, feat), jnp.float32),
            pltpu.VMEM((C, feat), jnp.float32),
            pltpu.SemaphoreType.DMA,
            pltpu.SemaphoreType.DMA,
            pltpu.SemaphoreType.DMA,
            pltpu.SemaphoreType.DMA,
            pltpu.SemaphoreType.DMA,
        ],
    )
    def k(src_h, adj_h, out_h, idx4, cb0, cb1, ob0, ob1,
          sidx, cs0, cs1, os0, os1):
        w = _wid()
        cbs = [cb0, cb1]
        css = [cs0, cs1]

        def chunk(kk, ob, os_, d, start):
            c = start + 32 * kk
            node0 = (d - 1) * PER + c * C
            pend = {0: pltpu.async_copy(src_h.at[idx4.at[kk, 0]], ob, css[0])}
            if d > 1:
                pend[1] = pltpu.async_copy(src_h.at[idx4.at[kk, 1]],
                                           cbs[1], css[1])
            pend[0].wait()
            for j in range(1, d):
                pend[j].wait()
                if j + 1 < d:
                    pend[j + 1] = pltpu.async_copy(
                        src_h.at[idx4.at[kk, j + 1]],
                        cbs[(j + 1) % 2], css[(j + 1) % 2])

                @pl.loop(0, C)
                def _(i, j=j):
                    for cc in range(nls):
                        sl = pl.ds(cc * 16, 16)
                        ob[i, sl] = ob[i, sl] + cbs[j % 2][i, sl]

            return pltpu.async_copy(ob, out_h.at[pl.ds(node0, C)], os_)

        for d in range(1, MAXD + 1):
            start = lax.rem(w + _ROT[d], 32)
            nch = jnp.where(start < 29, 4, 3).astype(jnp.int32)
            t0 = (d - 1) * NCH + start
            _prefetch_idx(adj_h, idx4, sidx, t0, nch)

            @pl.loop(0, 2)
            def _(p, d=d, start=start, nch=nch):
                o_a = chunk(2 * p, ob0, os0, d, start)
                more = 2 * p + 1 < nch

                @pl.when(more)
                def _(p=p, d=d, start=start):
                    o_b = chunk(2 * p + 1, ob1, os1, d, start)
                    o_a.wait()
                    o_b.wait()

                @pl.when(jnp.logical_not(more))
                def _():
                    o_a.wait()

    return k(src, adj2)


def _sc_gather_max(src, adj2, so, feat):
    nls = feat // 16

    @functools.partial(
        pl.kernel,
        out_type=jax.ShapeDtypeStruct((N, feat), jnp.float32),
        mesh=_mesh(),
        scratch_types=[
            pltpu.VMEM((4, 16, C), jnp.int32),
            pltpu.VMEM((C, feat), jnp.float32),
            pltpu.VMEM((C, feat), jnp.float32),
            pltpu.VMEM((C, feat), jnp.float32),
            pltpu.VMEM((C, feat), jnp.float32),
            pltpu.VMEM((C, feat), jnp.float32),
            pltpu.VMEM((C, feat), jnp.float32),
            pltpu.VMEM((2, feat), jnp.float32),
            pltpu.SemaphoreType.DMA,
            pltpu.SemaphoreType.DMA,
            pltpu.SemaphoreType.DMA,
            pltpu.SemaphoreType.DMA,
            pltpu.SemaphoreType.DMA,
            pltpu.SemaphoreType.DMA,
            pltpu.SemaphoreType.DMA,
        ],
    )
    def k(src_h, adj_h, so_h, out_h, idx4, cb0, cb1, sb0, sb1, ob0, ob1,
          so_v, sidx, cs0, cs1, ss0, ss1, os0, os1):
        w = _wid()
        cbs = [cb0, cb1]
        css = [cs0, cs1]
        pltpu.sync_copy(so_h, so_v)
        scs = [so_v[0, pl.ds(cc * 16, 16)] for cc in range(nls)]
        ofs = [so_v[1, pl.ds(cc * 16, 16)] for cc in range(nls)]

        def chunk(kk, sb, ss_, ob, os_, d, start):
            c = start + 32 * kk
            node0 = (d - 1) * PER + c * C
            sp = pltpu.async_copy(src_h.at[pl.ds(node0, C)], sb, ss_)
            pend = {0: pltpu.async_copy(src_h.at[idx4.at[kk, 0]],
                                        cbs[0], css[0])}
            if d > 1:
                pend[1] = pltpu.async_copy(src_h.at[idx4.at[kk, 1]],
                                           cbs[1], css[1])
            sp.wait()

            @pl.loop(0, C)
            def _(i):
                for cc in range(nls):
                    sl = pl.ds(cc * 16, 16)
                    ob[i, sl] = sb[i, sl] * scs[cc] + ofs[cc]

            for j in range(d):
                pend[j].wait()

                @pl.loop(0, C)
                def _(i, j=j):
                    for cc in range(nls):
                        sl = pl.ds(cc * 16, 16)
                        r = cbs[j % 2][i, sl] * scs[cc] + ofs[cc]
                        ob[i, sl] = jnp.maximum(ob[i, sl], r)

                if j + 2 < d:
                    pend[j + 2] = pltpu.async_copy(
                        src_h.at[idx4.at[kk, j + 2]], cbs[j % 2], css[j % 2])

            return pltpu.async_copy(ob, out_h.at[pl.ds(node0, C)], os_)

        for d in range(1, MAXD + 1):
            start = lax.rem(w + _ROT[d], 32)
            nch = jnp.where(start < 29, 4, 3).astype(jnp.int32)
            t0 = (d - 1) * NCH + start
            _prefetch_idx(adj_h, idx4, sidx, t0, nch)

            @pl.loop(0, 2)
            def _(p, d=d, start=start, nch=nch):
                o_a = chunk(2 * p, sb0, ss0, ob0, os0, d, start)
                more = 2 * p + 1 < nch

                @pl.when(more)
                def _(p=p, d=d, start=start):
                    o_b = chunk(2 * p + 1, sb1, ss1, ob1, os1, d, start)
                    o_a.wait()
                    o_b.wait()

                @pl.when(jnp.logical_not(more))
                def _():
                    o_a.wait()

    return k(src, adj2, so)


def _sc_segment(h2, mem, so):
    nls = 8

    @functools.partial(
        pl.kernel,
        out_type=(
            jax.ShapeDtypeStruct((2, NG, 128), jnp.float32),
            jax.ShapeDtypeStruct((2, NG, 128), jnp.float32),
            jax.ShapeDtypeStruct((2, NG, 16), jnp.float32),
        ),
        mesh=_mesh(),
        scratch_types=[
            pltpu.VMEM((64,), jnp.int32),
            pltpu.VMEM((64, 128), jnp.float32),
            pltpu.VMEM((NG, 128), jnp.float32),
            pltpu.VMEM((NG, 128), jnp.float32),
            pltpu.VMEM((NG, 16), jnp.float32),
            pltpu.VMEM((2, 128), jnp.float32),
            pltpu.VMEM((8, 128), jnp.float32),
            pltpu.VMEM((8, 16), jnp.float32),
            pltpu.VMEM_SHARED((16, NG, 128), jnp.float32),
            pltpu.VMEM_SHARED((16, NG, 128), jnp.float32),
            pltpu.VMEM_SHARED((16, NG, 16), jnp.float32),
        ],
    )
    def k(h2_h, mem_h, so_h, sum_o, max_o, cnt_o,
          memb, hbuf, sum_t, max_t, cnt_t, so_v, redb, cntr,
          sum_s, max_s, cnt_s):
        cid = lax.axis_index("c")
        sid = lax.axis_index("s")
        w = sid * 2 + cid
        pltpu.sync_copy(so_h, so_v)
        scs = [so_v[0, pl.ds(cc * 16, 16)] for cc in range(nls)]
        ofs = [so_v[1, pl.ds(cc * 16, 16)] for cc in range(nls)]
        zero = jnp.zeros((16,), jnp.float32)
        neg = jnp.full((16,), -3.4e38, jnp.float32)
        one0 = jnp.where(lax.iota(jnp.int32, 16) == 0, 1.0, 0.0
                         ).astype(jnp.float32)

        @pl.loop(0, NG)
        def _(r):
            for cc in range(nls):
                sl = pl.ds(cc * 16, 16)
                sum_t[r, sl] = zero
                max_t[r, sl] = neg
            cnt_t[r, :] = zero

        def group_body(i0):
            mvec = memb[pl.ds(i0, 16)]
            for ln in range(16):
                m = mvec[ln]
                i = i0 + ln
                for cc in range(nls):
                    sl = pl.ds(cc * 16, 16)
                    r = hbuf[i, sl] * scs[cc] + ofs[cc]
                    sum_t[m, sl] = sum_t[m, sl] + r
                    max_t[m, sl] = jnp.maximum(max_t[m, sl], r)
                cnt_t[m, :] = cnt_t[m, :] + one0

        nc = jnp.where(w < 26, 49, 48).astype(jnp.int32)

        @pl.loop(0, 49)
        def _(kk):
            @pl.when(kk < nc)
            def _(kk=kk):
                row0 = (w + 32 * kk) * 64
                pltpu.sync_copy(mem_h.at[pl.ds(row0, 64)], memb)
                pltpu.sync_copy(h2_h.at[pl.ds(row0, 64)], hbuf)

                @pl.loop(0, 64, step=16)
                def _(i0):
                    group_body(i0)

        @pl.when(w == 0)
        def _():
            pltpu.sync_copy(mem_h.at[pl.ds(99968, 32)], memb.at[pl.ds(0, 32)])
            pltpu.sync_copy(h2_h.at[pl.ds(99968, 32)], hbuf.at[pl.ds(0, 32)])

            @pl.loop(0, 32, step=16)
            def _(i0):
                group_body(i0)

        pltpu.sync_copy(sum_t, sum_s.at[sid])
        pltpu.sync_copy(max_t, max_s.at[sid])
        pltpu.sync_copy(cnt_t, cnt_s.at[sid])
        plsc.subcore_barrier()

        r0 = sid * 8
        for rr in range(8):
            for cc in range(nls):
                sum_t[rr, pl.ds(cc * 16, 16)] = zero
                max_t[rr, pl.ds(cc * 16, 16)] = neg
            cnt_t[rr, :] = zero

        @pl.loop(0, 16)
        def _(t):
            pltpu.sync_copy(sum_s.at[t, pl.ds(r0, 8), :], redb)
            for rr in range(8):
                for cc in range(nls):
                    sl = pl.ds(cc * 16, 16)
                    sum_t[rr, sl] = sum_t[rr, sl] + redb[rr, sl]
            pltpu.sync_copy(max_s.at[t, pl.ds(r0, 8), :], redb)
            for rr in range(8):
                for cc in range(nls):
                    sl = pl.ds(cc * 16, 16)
                    max_t[rr, sl] = jnp.maximum(max_t[rr, sl], redb[rr, sl])
            pltpu.sync_copy(cnt_s.at[t, pl.ds(r0, 8), :], cntr)
            for rr in range(8):
                cnt_t[rr, :] = cnt_t[rr, :] + cntr[rr, :]

        pltpu.sync_copy(sum_t.at[pl.ds(0, 8)], sum_o.at[cid, pl.ds(r0, 8), :])
        pltpu.sync_copy(max_t.at[pl.ds(0, 8)], max_o.at[cid, pl.ds(r0, 8), :])
        pltpu.sync_copy(cnt_t.at[pl.ds(0, 8)], cnt_o.at[cid, pl.ds(r0, 8), :])

    return k(h2, mem, so)


def _tc_gconv(neigh, selfx, wn, ws, b, g, bb, f_in):
    def body(n_ref, s_ref, wn_ref, ws_ref, b_ref, g_ref, bb_ref,
             h_ref, so_ref, acc_ref):
        d = pl.program_id(0)
        i = pl.program_id(1)

        @pl.when((d == 0) & (i == 0))
        def _():
            acc_ref[...] = jnp.zeros_like(acc_ref)

        h = jnp.dot(n_ref[...], wn_ref[0], preferred_element_type=jnp.float32)
        h = h + jnp.dot(s_ref[...], ws_ref[0],
                        preferred_element_type=jnp.float32)
        h = jnp.maximum(h + b_ref[0, 0], 0.0)
        h_ref[...] = jnp.concatenate(
            [h, jnp.zeros((1000, 64), jnp.float32)], axis=1)
        acc_ref[0, :] = acc_ref[0, :] + jnp.sum(h, axis=0)
        acc_ref[1, :] = acc_ref[1, :] + jnp.sum(h * h, axis=0)

        @pl.when((d == 9) & (i == 9))
        def _():
            mean = acc_ref[0, :] / N
            var = acc_ref[1, :] / N - mean * mean
            scale = g_ref[0] * lax.rsqrt(var + 1e-5)
            pad = jnp.zeros((64,), jnp.float32)
            so_ref[0, :] = jnp.concatenate([scale, pad])
            so_ref[1, :] = jnp.concatenate([bb_ref[0] - mean * scale, pad])

    return pl.pallas_call(
        body,
        grid=(10, 10),
        in_specs=[
            pl.BlockSpec((1000, f_in), lambda d, i: (d * 10 + i, 0)),
            pl.BlockSpec((1000, f_in), lambda d, i: (d * 10 + i, 0)),
            pl.BlockSpec((1, f_in, 64), lambda d, i: (d, 0, 0)),
            pl.BlockSpec((1, f_in, 64), lambda d, i: (d, 0, 0)),
            pl.BlockSpec((1, 1, 64), lambda d, i: (d, 0, 0)),
            pl.BlockSpec((1, 64), lambda d, i: (0, 0)),
            pl.BlockSpec((1, 64), lambda d, i: (0, 0)),
        ],
        out_specs=[
            pl.BlockSpec((1000, 128), lambda d, i: (d * 10 + i, 0)),
            pl.BlockSpec((2, 128), lambda d, i: (0, 0)),
        ],
        out_shape=[
            jax.ShapeDtypeStruct((N, 128), jnp.float32),
            jax.ShapeDtypeStruct((2, 128), jnp.float32),
        ],
        scratch_shapes=[pltpu.VMEM((2, 64), jnp.float32)],
    )(neigh, selfx, wn, ws, b.reshape(10, 1, 64), g.reshape(1, -1),
      bb.reshape(1, -1))


def _tc_dense(x2, w, b, g, bb):
    def body(x_ref, w_ref, b_ref, g_ref, bb_ref, h_ref, so_ref, acc_ref):
        i = pl.program_id(0)

        @pl.when(i == 0)
        def _():
            acc_ref[...] = jnp.zeros_like(acc_ref)

        h = jnp.dot(x_ref[...], w_ref[...], preferred_element_type=jnp.float32)
        h = jnp.maximum(h + b_ref[0], 0.0)
        h_ref[...] = h
        acc_ref[0, :] = acc_ref[0, :] + jnp.sum(h, axis=0)
        acc_ref[1, :] = acc_ref[1, :] + jnp.sum(h * h, axis=0)

        @pl.when(i == 99)
        def _():
            mean = acc_ref[0, :] / N
            var = acc_ref[1, :] / N - mean * mean
            scale = g_ref[0] * lax.rsqrt(var + 1e-5)
            so_ref[0, :] = scale
            so_ref[1, :] = bb_ref[0] - mean * scale

    return pl.pallas_call(
        body,
        grid=(100,),
        in_specs=[
            pl.BlockSpec((1000, 128), lambda i: (i, 0)),
            pl.BlockSpec((128, 128), lambda i: (0, 0)),
            pl.BlockSpec((1, 128), lambda i: (0, 0)),
            pl.BlockSpec((1, 128), lambda i: (0, 0)),
            pl.BlockSpec((1, 128), lambda i: (0, 0)),
        ],
        out_specs=[
            pl.BlockSpec((1000, 128), lambda i: (i, 0)),
            pl.BlockSpec((2, 128), lambda i: (0, 0)),
        ],
        out_shape=[
            jax.ShapeDtypeStruct((N, 128), jnp.float32),
            jax.ShapeDtypeStruct((2, 128), jnp.float32),
        ],
        scratch_shapes=[pltpu.VMEM((2, 128), jnp.float32)],
    )(x2, w, b.reshape(1, -1), g.reshape(1, -1), bb.reshape(1, -1))


def _tc_final(ssum, smax, scnt, w1, b1):
    def body(s_ref, m_ref, c_ref, w_ref, b_ref, o_ref):
        s = s_ref[0] + s_ref[1]
        m = jnp.maximum(m_ref[0], m_ref[1])
        cnt = c_ref[0, :, 0:1] + c_ref[1, :, 0:1]
        gg = jnp.concatenate([s / cnt, m], axis=1)
        o_ref[...] = jnp.dot(gg, w_ref[...],
                             preferred_element_type=jnp.float32) + b_ref[0]

    return pl.pallas_call(
        body,
        out_shape=jax.ShapeDtypeStruct((NG, 2), jnp.float32),
    )(ssum, smax, scnt, w1, b1.reshape(1, -1))


def kernel(node_features, deg_slice, membership, gc0_W, gc0_b, gc1_W, gc1_b,
           bn0_g, bn0_b, bn1_g, bn1_b, dense0_W, dense0_b, bn2_g, bn2_b,
           dense1_W, dense1_b, deg_adj_1, deg_adj_2, deg_adj_3, deg_adj_4,
           deg_adj_5, deg_adj_6, deg_adj_7, deg_adj_8, deg_adj_9, deg_adj_10):
    adjs = [deg_adj_1, deg_adj_2, deg_adj_3, deg_adj_4, deg_adj_5,
            deg_adj_6, deg_adj_7, deg_adj_8, deg_adj_9, deg_adj_10]
    parts = []
    for d, a in enumerate(adjs, 1):
        a32 = a.astype(jnp.int32)
        p = a32.reshape(NCH, C, d).transpose(0, 2, 1)
        p = jnp.pad(p, ((0, 0), (0, 16 - d), (0, 0)))
        parts.append(p)
    adj2 = jnp.concatenate(parts, axis=0)
    mem32 = membership.astype(jnp.int32)

    wn0, ws0 = gc0_W[1::2], gc0_W[2::2]
    b0 = gc0_b[1::2] + gc0_b[2::2]
    pad_w = ((0, 0), (0, 64), (0, 0))
    wn1 = jnp.pad(gc1_W[1::2], pad_w)
    ws1 = jnp.pad(gc1_W[2::2], pad_w)
    b1 = gc1_b[1::2] + gc1_b[2::2]
    d0w = jnp.pad(dense0_W, ((0, 64), (0, 0)))

    neigh0 = _sc_gather_sum(node_features, adj2, 128)
    h0, so0 = _tc_gconv(neigh0, node_features, wn0, ws0, b0, bn0_g, bn0_b, 128)
    x1 = _sc_gather_max(h0, adj2, so0, 128)
    neigh1 = _sc_gather_sum(x1, adj2, 128)
    h1, so1 = _tc_gconv(neigh1, x1, wn1, ws1, b1, bn1_g, bn1_b, 128)
    x2 = _sc_gather_max(h1, adj2, so1, 128)
    h2, so2 = _tc_dense(x2, d0w, dense0_b, bn2_g, bn2_b)
    ssum, smax, scnt = _sc_segment(h2, mem32, so2)
    return _tc_final(ssum, smax, scnt, dense1_W, dense1_b)


# speedup vs baseline: 3.4542x; 1.0124x over previous
import functools

import jax
import jax.numpy as jnp
from jax import lax
from jax.experimental import pallas as pl
from jax.experimental.pallas import tpu as pltpu
from jax.experimental.pallas import tpu_sc as plsc

N = 100000
PER = 10000
MAXD = 10
NG = 128
C = 80
NCH = 125
NW = 32

_ROT = [0] + [(29 - 3 * (10 - d)) % 32 for d in range(1, 11)]


def _mesh():
    return plsc.VectorSubcoreMesh(core_axis_name="c", subcore_axis_name="s")


def _wid():
    return lax.axis_index("s") * 2 + lax.axis_index("c")


def _prefetch_idx(adj_h, idx4, sidx, t0, nch):
    cps = [pltpu.async_copy(adj_h.at[t0 + 32 * kk], idx4.at[kk], sidx)
           for kk in range(3)]

    @pl.when(nch == 4)
    def _():
        pltpu.sync_copy(adj_h.at[t0 + 32 * 3], idx4.at[3])

    for cp in cps:
        cp.wait()


def _sc_gather_sum(src, adj2, feat):
    nls = feat // 16

    @functools.partial(
        pl.kernel,
        out_type=jax.ShapeDtypeStruct((N, feat), jnp.float32),
        mesh=_mesh(),
        scratch_types=[
            pltpu.VMEM((4, 16, C), jnp.int32),
            pltpu.VMEM((C, feat), jnp.float32),
            pltpu.VMEM((C, feat), jnp.float32),
            pltpu.VMEM((C, feat), jnp.float32),
            pltpu.VMEM((C, feat), jnp.float32),
            pltpu.SemaphoreType.DMA,
            pltpu.SemaphoreType.DMA,
            pltpu.SemaphoreType.DMA,
            pltpu.SemaphoreType.DMA,
            pltpu.SemaphoreType.DMA,
        ],
    )
    def k(src_h, adj_h, out_h, idx4, cb0, cb1, ob0, ob1,
          sidx, cs0, cs1, os0, os1):
        w = _wid()
        cbs = [cb0, cb1]
        css = [cs0, cs1]

        def chunk(kk, ob, os_, d, start):
            c = start + 32 * kk
            node0 = (d - 1) * PER + c * C
            pend = {0: pltpu.async_copy(src_h.at[idx4.at[kk, 0]], ob, css[0])}
            if d > 1:
                pend[1] = pltpu.async_copy(src_h.at[idx4.at[kk, 1]],
                                           cbs[1], css[1])
            pend[0].wait()
            for j in range(1, d):
                pend[j].wait()
                if j + 1 < d:
                    pend[j + 1] = pltpu.async_copy(
                        src_h.at[idx4.at[kk, j + 1]],
                        cbs[(j + 1) % 2], css[(j + 1) % 2])

                @pl.loop(0, C)
                def _(i, j=j):
                    for cc in range(nls):
                        sl = pl.ds(cc * 16, 16)
                        ob[i, sl] = ob[i, sl] + cbs[j % 2][i, sl]

            return pltpu.async_copy(ob, out_h.at[pl.ds(node0, C)], os_)

        for d in range(1, MAXD + 1):
            start = lax.rem(w + _ROT[d], 32)
            nch = jnp.where(start < 29, 4, 3).astype(jnp.int32)
            t0 = (d - 1) * NCH + start
            _prefetch_idx(adj_h, idx4, sidx, t0, nch)

            @pl.loop(0, 2)
            def _(p, d=d, start=start, nch=nch):
                o_a = chunk(2 * p, ob0, os0, d, start)
                more = 2 * p + 1 < nch

                @pl.when(more)
                def _(p=p, d=d, start=start):
                    o_b = chunk(2 * p + 1, ob1, os1, d, start)
                    o_a.wait()
                    o_b.wait()

                @pl.when(jnp.logical_not(more))
                def _():
                    o_a.wait()

    return k(src, adj2)


def _sc_gather_max(src, adj2, so, feat):
    nls = feat // 16

    @functools.partial(
        pl.kernel,
        out_type=jax.ShapeDtypeStruct((N, feat), jnp.float32),
        mesh=_mesh(),
        scratch_types=[
            pltpu.VMEM((4, 16, C), jnp.int32),
            pltpu.VMEM((C, feat), jnp.float32),
            pltpu.VMEM((C, feat), jnp.float32),
            pltpu.VMEM((C, feat), jnp.float32),
            pltpu.VMEM((C, feat), jnp.float32),
            pltpu.VMEM((C, feat), jnp.float32),
            pltpu.VMEM((C, feat), jnp.float32),
            pltpu.VMEM((2, feat), jnp.float32),
            pltpu.SemaphoreType.DMA,
            pltpu.SemaphoreType.DMA,
            pltpu.SemaphoreType.DMA,
            pltpu.SemaphoreType.DMA,
            pltpu.SemaphoreType.DMA,
            pltpu.SemaphoreType.DMA,
            pltpu.SemaphoreType.DMA,
        ],
    )
    def k(src_h, adj_h, so_h, out_h, idx4, cb0, cb1, sb0, sb1, ob0, ob1,
          so_v, sidx, cs0, cs1, ss0, ss1, os0, os1):
        w = _wid()
        cbs = [cb0, cb1]
        css = [cs0, cs1]
        pltpu.sync_copy(so_h, so_v)
        scs = [so_v[0, pl.ds(cc * 16, 16)] for cc in range(nls)]
        ofs = [so_v[1, pl.ds(cc * 16, 16)] for cc in range(nls)]

        def chunk(kk, sb, ss_, ob, os_, d, start):
            c = start + 32 * kk
            node0 = (d - 1) * PER + c * C
            sp = pltpu.async_copy(src_h.at[pl.ds(node0, C)], sb, ss_)
            pend = {0: pltpu.async_copy(src_h.at[idx4.at[kk, 0]],
                                        cbs[0], css[0])}
            if d > 1:
                pend[1] = pltpu.async_copy(src_h.at[idx4.at[kk, 1]],
                                           cbs[1], css[1])
            sp.wait()

            @pl.loop(0, C)
            def _(i):
                for cc in range(nls):
                    sl = pl.ds(cc * 16, 16)
                    ob[i, sl] = sb[i, sl] * scs[cc] + ofs[cc]

            for j in range(d):
                pend[j].wait()

                @pl.loop(0, C)
                def _(i, j=j):
                    for cc in range(nls):
                        sl = pl.ds(cc * 16, 16)
                        r = cbs[j % 2][i, sl] * scs[cc] + ofs[cc]
                        ob[i, sl] = jnp.maximum(ob[i, sl], r)

                if j + 2 < d:
                    pend[j + 2] = pltpu.async_copy(
                        src_h.at[idx4.at[kk, j + 2]], cbs[j % 2], css[j % 2])

            return pltpu.async_copy(ob, out_h.at[pl.ds(node0, C)], os_)

        for d in range(1, MAXD + 1):
            start = lax.rem(w + _ROT[d], 32)
            nch = jnp.where(start < 29, 4, 3).astype(jnp.int32)
            t0 = (d - 1) * NCH + start
            _prefetch_idx(adj_h, idx4, sidx, t0, nch)

            @pl.loop(0, 2)
            def _(p, d=d, start=start, nch=nch):
                o_a = chunk(2 * p, sb0, ss0, ob0, os0, d, start)
                more = 2 * p + 1 < nch

                @pl.when(more)
                def _(p=p, d=d, start=start):
                    o_b = chunk(2 * p + 1, sb1, ss1, ob1, os1, d, start)
                    o_a.wait()
                    o_b.wait()

                @pl.when(jnp.logical_not(more))
                def _():
                    o_a.wait()

    return k(src, adj2, so)


def _sc_segment(h2, mem, so):
    nls = 8

    @functools.partial(
        pl.kernel,
        out_type=(
            jax.ShapeDtypeStruct((2, NG, 128), jnp.float32),
            jax.ShapeDtypeStruct((2, NG, 128), jnp.float32),
            jax.ShapeDtypeStruct((2, NG, 16), jnp.float32),
        ),
        mesh=_mesh(),
        scratch_types=[
            pltpu.VMEM((128,), jnp.int32),
            pltpu.VMEM((128,), jnp.int32),
            pltpu.VMEM((128, 128), jnp.float32),
            pltpu.VMEM((128, 128), jnp.float32),
            pltpu.VMEM((NG, 128), jnp.float32),
            pltpu.VMEM((NG, 128), jnp.float32),
            pltpu.VMEM((NG, 16), jnp.float32),
            pltpu.VMEM((2, 128), jnp.float32),
            pltpu.VMEM((8, 128), jnp.float32),
            pltpu.VMEM((8, 16), jnp.float32),
            pltpu.SemaphoreType.DMA,
            pltpu.SemaphoreType.DMA,
            pltpu.SemaphoreType.DMA,
            pltpu.SemaphoreType.DMA,
            pltpu.VMEM_SHARED((16, NG, 128), jnp.float32),
            pltpu.VMEM_SHARED((16, NG, 128), jnp.float32),
            pltpu.VMEM_SHARED((16, NG, 16), jnp.float32),
        ],
    )
    def k(h2_h, mem_h, so_h, sum_o, max_o, cnt_o,
          memb0, memb1, hbuf0, hbuf1, sum_t, max_t, cnt_t, so_v, redb, cntr,
          sem0m, sem0h, sem1m, sem1h, sum_s, max_s, cnt_s):
        cid = lax.axis_index("c")
        sid = lax.axis_index("s")
        w = sid * 2 + cid
        pltpu.sync_copy(so_h, so_v)
        scs = [so_v[0, pl.ds(cc * 16, 16)] for cc in range(nls)]
        ofs = [so_v[1, pl.ds(cc * 16, 16)] for cc in range(nls)]
        zero = jnp.zeros((16,), jnp.float32)
        neg = jnp.full((16,), -3.4e38, jnp.float32)
        one0 = jnp.where(lax.iota(jnp.int32, 16) == 0, 1.0, 0.0
                         ).astype(jnp.float32)

        @pl.loop(0, NG)
        def _(r):
            for cc in range(nls):
                sl = pl.ds(cc * 16, 16)
                sum_t[r, sl] = zero
                max_t[r, sl] = neg
            cnt_t[r, :] = zero

        def group_body(i0, memb, hbuf):
            mvec = memb[pl.ds(i0, 16)]
            for ln in range(16):
                m = mvec[ln]
                i = i0 + ln
                for cc in range(nls):
                    sl = pl.ds(cc * 16, 16)
                    r = hbuf[i, sl] * scs[cc] + ofs[cc]
                    sum_t[m, sl] = sum_t[m, sl] + r
                    max_t[m, sl] = jnp.maximum(max_t[m, sl], r)
                cnt_t[m, :] = cnt_t[m, :] + one0

        nc = jnp.where(w < 13, 25, 24).astype(jnp.int32)

        @pl.loop(0, 25)
        def _(kk):
            @pl.when(kk < nc)
            def _(kk=kk):
                row0 = (w + 32 * kk) * 128
                pltpu.sync_copy(mem_h.at[pl.ds(row0, 128)], memb0)
                pltpu.sync_copy(h2_h.at[pl.ds(row0, 128)], hbuf0)

                @pl.loop(0, 128, step=16)
                def _(i0):
                    group_body(i0, memb0, hbuf0)

        @pl.when(w == 0)
        def _():
            pltpu.sync_copy(mem_h.at[pl.ds(99968, 32)],
                            memb0.at[pl.ds(0, 32)])
            pltpu.sync_copy(h2_h.at[pl.ds(99968, 32)],
                            hbuf0.at[pl.ds(0, 32)])

            @pl.loop(0, 32, step=16)
            def _(i0):
                group_body(i0, memb0, hbuf0)

        pltpu.sync_copy(sum_t, sum_s.at[sid])
        pltpu.sync_copy(max_t, max_s.at[sid])
        pltpu.sync_copy(cnt_t, cnt_s.at[sid])
        plsc.subcore_barrier()

        r0 = sid * 8
        for rr in range(8):
            for cc in range(nls):
                sum_t[rr, pl.ds(cc * 16, 16)] = zero
                max_t[rr, pl.ds(cc * 16, 16)] = neg
            cnt_t[rr, :] = zero

        @pl.loop(0, 16)
        def _(t):
            pltpu.sync_copy(sum_s.at[t, pl.ds(r0, 8), :], redb)
            for rr in range(8):
                for cc in range(nls):
                    sl = pl.ds(cc * 16, 16)
                    sum_t[rr, sl] = sum_t[rr, sl] + redb[rr, sl]
            pltpu.sync_copy(max_s.at[t, pl.ds(r0, 8), :], redb)
            for rr in range(8):
                for cc in range(nls):
                    sl = pl.ds(cc * 16, 16)
                    max_t[rr, sl] = jnp.maximum(max_t[rr, sl], redb[rr, sl])
            pltpu.sync_copy(cnt_s.at[t, pl.ds(r0, 8), :], cntr)
            for rr in range(8):
                cnt_t[rr, :] = cnt_t[rr, :] + cntr[rr, :]

        pltpu.sync_copy(sum_t.at[pl.ds(0, 8)], sum_o.at[cid, pl.ds(r0, 8), :])
        pltpu.sync_copy(max_t.at[pl.ds(0, 8)], max_o.at[cid, pl.ds(r0, 8), :])
        pltpu.sync_copy(cnt_t.at[pl.ds(0, 8)], cnt_o.at[cid, pl.ds(r0, 8), :])

    return k(h2, mem, so)


def _tc_gconv(neigh, selfx, wn, ws, b, g, bb, f_in):
    def body(n_ref, s_ref, wn_ref, ws_ref, b_ref, g_ref, bb_ref,
             h_ref, so_ref, acc_ref):
        d = pl.program_id(0)
        i = pl.program_id(1)

        @pl.when((d == 0) & (i == 0))
        def _():
            acc_ref[...] = jnp.zeros_like(acc_ref)

        h = jnp.dot(n_ref[...], wn_ref[0], preferred_element_type=jnp.float32)
        h = h + jnp.dot(s_ref[...], ws_ref[0],
                        preferred_element_type=jnp.float32)
        h = jnp.maximum(h + b_ref[0, 0], 0.0)
        h_ref[...] = jnp.concatenate(
            [h, jnp.zeros((1000, 64), jnp.float32)], axis=1)
        acc_ref[0, :] = acc_ref[0, :] + jnp.sum(h, axis=0)
        acc_ref[1, :] = acc_ref[1, :] + jnp.sum(h * h, axis=0)

        @pl.when((d == 9) & (i == 9))
        def _():
            mean = acc_ref[0, :] / N
            var = acc_ref[1, :] / N - mean * mean
            scale = g_ref[0] * lax.rsqrt(var + 1e-5)
            pad = jnp.zeros((64,), jnp.float32)
            so_ref[0, :] = jnp.concatenate([scale, pad])
            so_ref[1, :] = jnp.concatenate([bb_ref[0] - mean * scale, pad])

    return pl.pallas_call(
        body,
        grid=(10, 10),
        in_specs=[
            pl.BlockSpec((1000, f_in), lambda d, i: (d * 10 + i, 0)),
            pl.BlockSpec((1000, f_in), lambda d, i: (d * 10 + i, 0)),
            pl.BlockSpec((1, f_in, 64), lambda d, i: (d, 0, 0)),
            pl.BlockSpec((1, f_in, 64), lambda d, i: (d, 0, 0)),
            pl.BlockSpec((1, 1, 64), lambda d, i: (d, 0, 0)),
            pl.BlockSpec((1, 64), lambda d, i: (0, 0)),
            pl.BlockSpec((1, 64), lambda d, i: (0, 0)),
        ],
        out_specs=[
            pl.BlockSpec((1000, 128), lambda d, i: (d * 10 + i, 0)),
            pl.BlockSpec((2, 128), lambda d, i: (0, 0)),
        ],
        out_shape=[
            jax.ShapeDtypeStruct((N, 128), jnp.float32),
            jax.ShapeDtypeStruct((2, 128), jnp.float32),
        ],
        scratch_shapes=[pltpu.VMEM((2, 64), jnp.float32)],
    )(neigh, selfx, wn, ws, b.reshape(10, 1, 64), g.reshape(1, -1),
      bb.reshape(1, -1))


def _tc_dense(x2, w, b, g, bb):
    def body(x_ref, w_ref, b_ref, g_ref, bb_ref, h_ref, so_ref, acc_ref):
        i = pl.program_id(0)

        @pl.when(i == 0)
        def _():
            acc_ref[...] = jnp.zeros_like(acc_ref)

        h = jnp.dot(x_ref[...], w_ref[...], preferred_element_type=jnp.float32)
        h = jnp.maximum(h + b_ref[0], 0.0)
        h_ref[...] = h
        acc_ref[0, :] = acc_ref[0, :] + jnp.sum(h, axis=0)
        acc_ref[1, :] = acc_ref[1, :] + jnp.sum(h * h, axis=0)

        @pl.when(i == 99)
        def _():
            mean = acc_ref[0, :] / N
            var = acc_ref[1, :] / N - mean * mean
            scale = g_ref[0] * lax.rsqrt(var + 1e-5)
            so_ref[0, :] = scale
            so_ref[1, :] = bb_ref[0] - mean * scale

    return pl.pallas_call(
        body,
        grid=(100,),
        in_specs=[
            pl.BlockSpec((1000, 128), lambda i: (i, 0)),
            pl.BlockSpec((128, 128), lambda i: (0, 0)),
            pl.BlockSpec((1, 128), lambda i: (0, 0)),
            pl.BlockSpec((1, 128), lambda i: (0, 0)),
            pl.BlockSpec((1, 128), lambda i: (0, 0)),
        ],
        out_specs=[
            pl.BlockSpec((1000, 128), lambda i: (i, 0)),
            pl.BlockSpec((2, 128), lambda i: (0, 0)),
        ],
        out_shape=[
            jax.ShapeDtypeStruct((N, 128), jnp.float32),
            jax.ShapeDtypeStruct((2, 128), jnp.float32),
        ],
        scratch_shapes=[pltpu.VMEM((2, 128), jnp.float32)],
    )(x2, w, b.reshape(1, -1), g.reshape(1, -1), bb.reshape(1, -1))


def _tc_final(ssum, smax, scnt, w1, b1):
    def body(s_ref, m_ref, c_ref, w_ref, b_ref, o_ref):
        s = s_ref[0] + s_ref[1]
        m = jnp.maximum(m_ref[0], m_ref[1])
        cnt = c_ref[0, :, 0:1] + c_ref[1, :, 0:1]
        gg = jnp.concatenate([s / cnt, m], axis=1)
        o_ref[...] = jnp.dot(gg, w_ref[...],
                             preferred_element_type=jnp.float32) + b_ref[0]

    return pl.pallas_call(
        body,
        out_shape=jax.ShapeDtypeStruct((NG, 2), jnp.float32),
    )(ssum, smax, scnt, w1, b1.reshape(1, -1))


def kernel(node_features, deg_slice, membership, gc0_W, gc0_b, gc1_W, gc1_b,
           bn0_g, bn0_b, bn1_g, bn1_b, dense0_W, dense0_b, bn2_g, bn2_b,
           dense1_W, dense1_b, deg_adj_1, deg_adj_2, deg_adj_3, deg_adj_4,
           deg_adj_5, deg_adj_6, deg_adj_7, deg_adj_8, deg_adj_9, deg_adj_10):
    adjs = [deg_adj_1, deg_adj_2, deg_adj_3, deg_adj_4, deg_adj_5,
            deg_adj_6, deg_adj_7, deg_adj_8, deg_adj_9, deg_adj_10]
    parts = []
    for d, a in enumerate(adjs, 1):
        a32 = a.astype(jnp.int32)
        p = a32.reshape(NCH, C, d).transpose(0, 2, 1)
        p = jnp.pad(p, ((0, 0), (0, 16 - d), (0, 0)))
        parts.append(p)
    adj2 = jnp.concatenate(parts, axis=0)
    mem32 = membership.astype(jnp.int32)

    wn0, ws0 = gc0_W[1::2], gc0_W[2::2]
    b0 = gc0_b[1::2] + gc0_b[2::2]
    pad_w = ((0, 0), (0, 64), (0, 0))
    wn1 = jnp.pad(gc1_W[1::2], pad_w)
    ws1 = jnp.pad(gc1_W[2::2], pad_w)
    b1 = gc1_b[1::2] + gc1_b[2::2]
    d0w = jnp.pad(dense0_W, ((0, 64), (0, 0)))

    neigh0 = _sc_gather_sum(node_features, adj2, 128)
    h0, so0 = _tc_gconv(neigh0, node_features, wn0, ws0, b0, bn0_g, bn0_b, 128)
    x1 = _sc_gather_max(h0, adj2, so0, 128)
    neigh1 = _sc_gather_sum(x1, adj2, 128)
    h1, so1 = _tc_gconv(neigh1, x1, wn1, ws1, b1, bn1_g, bn1_b, 128)
    x2 = _sc_gather_max(h1, adj2, so1, 128)
    h2, so2 = _tc_dense(x2, d0w, dense0_b, bn2_g, bn2_b)
    ssum, smax, scnt = _sc_segment(h2, mem32, so2)
    return _tc_final(ssum, smax, scnt, dense1_W, dense1_b)


# speedup vs baseline: 3.4807x; 1.0077x over previous
import functools

import jax
import jax.numpy as jnp
from jax import lax
from jax.experimental import pallas as pl
from jax.experimental.pallas import tpu as pltpu
from jax.experimental.pallas import tpu_sc as plsc

N = 100000
PER = 10000
MAXD = 10
NG = 128
C = 80
NCH = 125
NW = 32

_ROT = [0] + [(29 - 3 * (10 - d)) % 32 for d in range(1, 11)]


def _mesh():
    return plsc.VectorSubcoreMesh(core_axis_name="c", subcore_axis_name="s")


def _wid():
    return lax.axis_index("s") * 2 + lax.axis_index("c")


def _prefetch_idx(adj_h, idx4, sidx, t0, nch):
    cps = [pltpu.async_copy(adj_h.at[t0 + 32 * kk], idx4.at[kk], sidx)
           for kk in range(3)]

    @pl.when(nch == 4)
    def _():
        pltpu.sync_copy(adj_h.at[t0 + 32 * 3], idx4.at[3])

    for cp in cps:
        cp.wait()


def _sc_gather_sum(src, adj2, feat):
    nls = feat // 16

    @functools.partial(
        pl.kernel,
        out_type=jax.ShapeDtypeStruct((N, feat), jnp.float32),
        mesh=_mesh(),
        scratch_types=[
            pltpu.VMEM((4, 16, C), jnp.int32),
            pltpu.VMEM((C, feat), jnp.float32),
            pltpu.VMEM((C, feat), jnp.float32),
            pltpu.VMEM((C, feat), jnp.float32),
            pltpu.VMEM((C, feat), jnp.float32),
            pltpu.SemaphoreType.DMA,
            pltpu.SemaphoreType.DMA,
            pltpu.SemaphoreType.DMA,
            pltpu.SemaphoreType.DMA,
            pltpu.SemaphoreType.DMA,
        ],
    )
    def k(src_h, adj_h, out_h, idx4, cb0, cb1, ob0, ob1,
          sidx, cs0, cs1, os0, os1):
        w = _wid()
        cbs = [cb0, cb1]
        css = [cs0, cs1]

        def chunk(kk, ob, os_, d, start):
            c = start + 32 * kk
            node0 = (d - 1) * PER + c * C
            pend = {0: pltpu.async_copy(src_h.at[idx4.at[kk, 0]], ob, css[0])}
            if d > 1:
                pend[1] = pltpu.async_copy(src_h.at[idx4.at[kk, 1]],
                                           cbs[1], css[1])
            pend[0].wait()
            for j in range(1, d):
                pend[j].wait()
                if j + 1 < d:
                    pend[j + 1] = pltpu.async_copy(
                        src_h.at[idx4.at[kk, j + 1]],
                        cbs[(j + 1) % 2], css[(j + 1) % 2])

                @pl.loop(0, C)
                def _(i, j=j):
                    for cc in range(nls):
                        sl = pl.ds(cc * 16, 16)
                        ob[i, sl] = ob[i, sl] + cbs[j % 2][i, sl]

            return pltpu.async_copy(ob, out_h.at[pl.ds(node0, C)], os_)

        for d in range(1, MAXD + 1):
            start = lax.rem(w + _ROT[d], 32)
            nch = jnp.where(start < 29, 4, 3).astype(jnp.int32)
            t0 = (d - 1) * NCH + start
            _prefetch_idx(adj_h, idx4, sidx, t0, nch)

            @pl.loop(0, 2)
            def _(p, d=d, start=start, nch=nch):
                o_a = chunk(2 * p, ob0, os0, d, start)
                more = 2 * p + 1 < nch

                @pl.when(more)
                def _(p=p, d=d, start=start):
                    o_b = chunk(2 * p + 1, ob1, os1, d, start)
                    o_a.wait()
                    o_b.wait()

                @pl.when(jnp.logical_not(more))
                def _():
                    o_a.wait()

    return k(src, adj2)


def _sc_gather_max(src, adj2, so, feat):
    nls = feat // 16

    @functools.partial(
        pl.kernel,
        out_type=jax.ShapeDtypeStruct((N, feat), jnp.float32),
        mesh=_mesh(),
        scratch_types=[
            pltpu.VMEM((4, 16, C), jnp.int32),
            pltpu.VMEM((C, feat), jnp.float32),
            pltpu.VMEM((C, feat), jnp.float32),
            pltpu.VMEM((C, feat), jnp.float32),
            pltpu.VMEM((C, feat), jnp.float32),
            pltpu.VMEM((C, feat), jnp.float32),
            pltpu.VMEM((C, feat), jnp.float32),
            pltpu.VMEM((2, feat), jnp.float32),
            pltpu.SemaphoreType.DMA,
            pltpu.SemaphoreType.DMA,
            pltpu.SemaphoreType.DMA,
            pltpu.SemaphoreType.DMA,
            pltpu.SemaphoreType.DMA,
            pltpu.SemaphoreType.DMA,
            pltpu.SemaphoreType.DMA,
        ],
    )
    def k(src_h, adj_h, so_h, out_h, idx4, cb0, cb1, sb0, sb1, ob0, ob1,
          so_v, sidx, cs0, cs1, ss0, ss1, os0, os1):
        w = _wid()
        cbs = [cb0, cb1]
        css = [cs0, cs1]
        pltpu.sync_copy(so_h, so_v)
        scs = [so_v[0, pl.ds(cc * 16, 16)] for cc in range(nls)]
        ofs = [so_v[1, pl.ds(cc * 16, 16)] for cc in range(nls)]

        def chunk(kk, sb, ss_, ob, os_, d, start):
            c = start + 32 * kk
            node0 = (d - 1) * PER + c * C
            sp = pltpu.async_copy(src_h.at[pl.ds(node0, C)], sb, ss_)
            pend = {0: pltpu.async_copy(src_h.at[idx4.at[kk, 0]],
                                        cbs[0], css[0])}
            if d > 1:
                pend[1] = pltpu.async_copy(src_h.at[idx4.at[kk, 1]],
                                           cbs[1], css[1])
            sp.wait()

            @pl.loop(0, C)
            def _(i):
                for cc in range(nls):
                    sl = pl.ds(cc * 16, 16)
                    ob[i, sl] = sb[i, sl] * scs[cc] + ofs[cc]

            for j in range(d):
                pend[j].wait()

                @pl.loop(0, C)
                def _(i, j=j):
                    for cc in range(nls):
                        sl = pl.ds(cc * 16, 16)
                        r = cbs[j % 2][i, sl] * scs[cc] + ofs[cc]
                        ob[i, sl] = jnp.maximum(ob[i, sl], r)

                if j + 2 < d:
                    pend[j + 2] = pltpu.async_copy(
                        src_h.at[idx4.at[kk, j + 2]], cbs[j % 2], css[j % 2])

            return pltpu.async_copy(ob, out_h.at[pl.ds(node0, C)], os_)

        for d in range(1, MAXD + 1):
            start = lax.rem(w + _ROT[d], 32)
            nch = jnp.where(start < 29, 4, 3).astype(jnp.int32)
            t0 = (d - 1) * NCH + start
            _prefetch_idx(adj_h, idx4, sidx, t0, nch)

            @pl.loop(0, 2)
            def _(p, d=d, start=start, nch=nch):
                o_a = chunk(2 * p, sb0, ss0, ob0, os0, d, start)
                more = 2 * p + 1 < nch

                @pl.when(more)
                def _(p=p, d=d, start=start):
                    o_b = chunk(2 * p + 1, sb1, ss1, ob1, os1, d, start)
                    o_a.wait()
                    o_b.wait()

                @pl.when(jnp.logical_not(more))
                def _():
                    o_a.wait()

    return k(src, adj2, so)


def _sc_segment(h2, mem, so):
    nls = 8

    @functools.partial(
        pl.kernel,
        out_type=jax.ShapeDtypeStruct((2, NG, 128), jnp.float32),
        mesh=_mesh(),
        scratch_types=[
            pltpu.VMEM((128,), jnp.int32),
            pltpu.VMEM((128, 128), jnp.float32),
            pltpu.VMEM((NG, 128), jnp.float32),
            pltpu.VMEM((2, 128), jnp.float32),
            pltpu.VMEM((8, 128), jnp.float32),
            pltpu.VMEM_SHARED((16, NG, 128), jnp.float32),
        ],
    )
    def k(h2_h, mem_h, so_h, max_o,
          memb0, hbuf0, max_t, so_v, redb, max_s):
        cid = lax.axis_index("c")
        sid = lax.axis_index("s")
        w = sid * 2 + cid
        pltpu.sync_copy(so_h, so_v)
        scs = [so_v[0, pl.ds(cc * 16, 16)] for cc in range(nls)]
        ofs = [so_v[1, pl.ds(cc * 16, 16)] for cc in range(nls)]
        neg = jnp.full((16,), -3.4e38, jnp.float32)

        @pl.loop(0, NG)
        def _(r):
            for cc in range(nls):
                max_t[r, pl.ds(cc * 16, 16)] = neg

        def group_body(i0, memb, hbuf):
            mvec = memb[pl.ds(i0, 16)]
            for ln in range(16):
                m = mvec[ln]
                i = i0 + ln
                for cc in range(nls):
                    sl = pl.ds(cc * 16, 16)
                    r = hbuf[i, sl] * scs[cc] + ofs[cc]
                    max_t[m, sl] = jnp.maximum(max_t[m, sl], r)

        nc = jnp.where(w < 13, 25, 24).astype(jnp.int32)

        @pl.loop(0, 25)
        def _(kk):
            @pl.when(kk < nc)
            def _(kk=kk):
                row0 = (w + 32 * kk) * 128
                pltpu.sync_copy(mem_h.at[pl.ds(row0, 128)], memb0)
                pltpu.sync_copy(h2_h.at[pl.ds(row0, 128)], hbuf0)

                @pl.loop(0, 128, step=16)
                def _(i0):
                    group_body(i0, memb0, hbuf0)

        @pl.when(w == 0)
        def _():
            pltpu.sync_copy(mem_h.at[pl.ds(99968, 32)],
                            memb0.at[pl.ds(0, 32)])
            pltpu.sync_copy(h2_h.at[pl.ds(99968, 32)],
                            hbuf0.at[pl.ds(0, 32)])

            @pl.loop(0, 32, step=16)
            def _(i0):
                group_body(i0, memb0, hbuf0)

        pltpu.sync_copy(max_t, max_s.at[sid])
        plsc.subcore_barrier()

        r0 = sid * 8
        for rr in range(8):
            for cc in range(nls):
                max_t[rr, pl.ds(cc * 16, 16)] = neg

        @pl.loop(0, 16)
        def _(t):
            pltpu.sync_copy(max_s.at[t, pl.ds(r0, 8), :], redb)
            for rr in range(8):
                for cc in range(nls):
                    sl = pl.ds(cc * 16, 16)
                    max_t[rr, sl] = jnp.maximum(max_t[rr, sl], redb[rr, sl])

        pltpu.sync_copy(max_t.at[pl.ds(0, 8)], max_o.at[cid, pl.ds(r0, 8), :])

    return k(h2, mem, so)


def _tc_gconv(neigh, selfx, wn, ws, b, g, bb, f_in):
    def body(n_ref, s_ref, wn_ref, ws_ref, b_ref, g_ref, bb_ref,
             h_ref, so_ref, acc_ref):
        d = pl.program_id(0)
        i = pl.program_id(1)

        @pl.when((d == 0) & (i == 0))
        def _():
            acc_ref[...] = jnp.zeros_like(acc_ref)

        h = jnp.dot(n_ref[...], wn_ref[0], preferred_element_type=jnp.float32)
        h = h + jnp.dot(s_ref[...], ws_ref[0],
                        preferred_element_type=jnp.float32)
        h = jnp.maximum(h + b_ref[0, 0], 0.0)
        h_ref[...] = jnp.concatenate(
            [h, jnp.zeros((1000, 64), jnp.float32)], axis=1)
        acc_ref[0, :] = acc_ref[0, :] + jnp.sum(h, axis=0)
        acc_ref[1, :] = acc_ref[1, :] + jnp.sum(h * h, axis=0)

        @pl.when((d == 9) & (i == 9))
        def _():
            mean = acc_ref[0, :] / N
            var = acc_ref[1, :] / N - mean * mean
            scale = g_ref[0] * lax.rsqrt(var + 1e-5)
            pad = jnp.zeros((64,), jnp.float32)
            so_ref[0, :] = jnp.concatenate([scale, pad])
            so_ref[1, :] = jnp.concatenate([bb_ref[0] - mean * scale, pad])

    return pl.pallas_call(
        body,
        grid=(10, 10),
        in_specs=[
            pl.BlockSpec((1000, f_in), lambda d, i: (d * 10 + i, 0)),
            pl.BlockSpec((1000, f_in), lambda d, i: (d * 10 + i, 0)),
            pl.BlockSpec((1, f_in, 64), lambda d, i: (d, 0, 0)),
            pl.BlockSpec((1, f_in, 64), lambda d, i: (d, 0, 0)),
            pl.BlockSpec((1, 1, 64), lambda d, i: (d, 0, 0)),
            pl.BlockSpec((1, 64), lambda d, i: (0, 0)),
            pl.BlockSpec((1, 64), lambda d, i: (0, 0)),
        ],
        out_specs=[
            pl.BlockSpec((1000, 128), lambda d, i: (d * 10 + i, 0)),
            pl.BlockSpec((2, 128), lambda d, i: (0, 0)),
        ],
        out_shape=[
            jax.ShapeDtypeStruct((N, 128), jnp.float32),
            jax.ShapeDtypeStruct((2, 128), jnp.float32),
        ],
        scratch_shapes=[pltpu.VMEM((2, 64), jnp.float32)],
    )(neigh, selfx, wn, ws, b.reshape(10, 1, 64), g.reshape(1, -1),
      bb.reshape(1, -1))


def _tc_dense(x2, w, b, g, bb, mem):
    def body(x_ref, w_ref, b_ref, g_ref, bb_ref, m_ref,
             h_ref, so_ref, seg_ref, cnt_ref, acc_ref, sacc_ref, cacc_ref):
        i = pl.program_id(0)

        @pl.when(i == 0)
        def _():
            acc_ref[...] = jnp.zeros_like(acc_ref)
            sacc_ref[...] = jnp.zeros_like(sacc_ref)
            cacc_ref[...] = jnp.zeros_like(cacc_ref)

        h = jnp.dot(x_ref[...], w_ref[...], preferred_element_type=jnp.float32)
        h = jnp.maximum(h + b_ref[0], 0.0)
        h_ref[...] = h
        acc_ref[0, :] = acc_ref[0, :] + jnp.sum(h, axis=0)
        acc_ref[1, :] = acc_ref[1, :] + jnp.sum(h * h, axis=0)

        seg_ids = jax.lax.broadcasted_iota(jnp.int32, (1000, NG), 1)
        onehot = jnp.where(m_ref[0, 0][:, None] == seg_ids, 1.0, 0.0
                           ).astype(jnp.float32)
        sacc_ref[...] = sacc_ref[...] + jax.lax.dot_general(
            onehot, h, (((0,), (0,)), ((), ())),
            preferred_element_type=jnp.float32)
        cacc_ref[0, :] = cacc_ref[0, :] + jnp.sum(onehot, axis=0)

        @pl.when(i == 99)
        def _():
            mean = acc_ref[0, :] / N
            var = acc_ref[1, :] / N - mean * mean
            scale = g_ref[0] * lax.rsqrt(var + 1e-5)
            so_ref[0, :] = scale
            so_ref[1, :] = bb_ref[0] - mean * scale
            seg_ref[...] = sacc_ref[...]
            cnt_ref[...] = cacc_ref[...]

    return pl.pallas_call(
        body,
        grid=(100,),
        in_specs=[
            pl.BlockSpec((1000, 128), lambda i: (i, 0)),
            pl.BlockSpec((128, 128), lambda i: (0, 0)),
            pl.BlockSpec((1, 128), lambda i: (0, 0)),
            pl.BlockSpec((1, 128), lambda i: (0, 0)),
            pl.BlockSpec((1, 128), lambda i: (0, 0)),
            pl.BlockSpec((1, 1, 1000), lambda i: (i, 0, 0)),
        ],
        out_specs=[
            pl.BlockSpec((1000, 128), lambda i: (i, 0)),
            pl.BlockSpec((2, 128), lambda i: (0, 0)),
            pl.BlockSpec((NG, 128), lambda i: (0, 0)),
            pl.BlockSpec((1, NG), lambda i: (0, 0)),
        ],
        out_shape=[
            jax.ShapeDtypeStruct((N, 128), jnp.float32),
            jax.ShapeDtypeStruct((2, 128), jnp.float32),
            jax.ShapeDtypeStruct((NG, 128), jnp.float32),
            jax.ShapeDtypeStruct((1, NG), jnp.float32),
        ],
        scratch_shapes=[pltpu.VMEM((2, 128), jnp.float32),
                        pltpu.VMEM((NG, 128), jnp.float32),
                        pltpu.VMEM((1, NG), jnp.float32)],
    )(x2, w, b.reshape(1, -1), g.reshape(1, -1), bb.reshape(1, -1),
      mem.reshape(100, 1, 1000))


def _tc_final(ssum, scnt, so, smax, w1, b1):
    def body(s_ref, c_ref, so_ref, m_ref, w_ref, b_ref, o_ref):
        m = jnp.maximum(m_ref[0], m_ref[1])
        mean = s_ref[...] / c_ref[0][:, None]
        mean = mean * so_ref[0][None, :] + so_ref[1][None, :]
        gg = jnp.concatenate([mean, m], axis=1)
        o_ref[...] = jnp.dot(gg, w_ref[...],
                             preferred_element_type=jnp.float32) + b_ref[0]

    return pl.pallas_call(
        body,
        out_shape=jax.ShapeDtypeStruct((NG, 2), jnp.float32),
    )(ssum, scnt, so, smax, w1, b1.reshape(1, -1))


def kernel(node_features, deg_slice, membership, gc0_W, gc0_b, gc1_W, gc1_b,
           bn0_g, bn0_b, bn1_g, bn1_b, dense0_W, dense0_b, bn2_g, bn2_b,
           dense1_W, dense1_b, deg_adj_1, deg_adj_2, deg_adj_3, deg_adj_4,
           deg_adj_5, deg_adj_6, deg_adj_7, deg_adj_8, deg_adj_9, deg_adj_10):
    adjs = [deg_adj_1, deg_adj_2, deg_adj_3, deg_adj_4, deg_adj_5,
            deg_adj_6, deg_adj_7, deg_adj_8, deg_adj_9, deg_adj_10]
    parts = []
    for d, a in enumerate(adjs, 1):
        a32 = a.astype(jnp.int32)
        p = a32.reshape(NCH, C, d).transpose(0, 2, 1)
        p = jnp.pad(p, ((0, 0), (0, 16 - d), (0, 0)))
        parts.append(p)
    adj2 = jnp.concatenate(parts, axis=0)
    mem32 = membership.astype(jnp.int32)

    wn0, ws0 = gc0_W[1::2], gc0_W[2::2]
    b0 = gc0_b[1::2] + gc0_b[2::2]
    pad_w = ((0, 0), (0, 64), (0, 0))
    wn1 = jnp.pad(gc1_W[1::2], pad_w)
    ws1 = jnp.pad(gc1_W[2::2], pad_w)
    b1 = gc1_b[1::2] + gc1_b[2::2]
    d0w = jnp.pad(dense0_W, ((0, 64), (0, 0)))

    neigh0 = _sc_gather_sum(node_features, adj2, 128)
    h0, so0 = _tc_gconv(neigh0, node_features, wn0, ws0, b0, bn0_g, bn0_b, 128)
    x1 = _sc_gather_max(h0, adj2, so0, 128)
    neigh1 = _sc_gather_sum(x1, adj2, 128)
    h1, so1 = _tc_gconv(neigh1, x1, wn1, ws1, b1, bn1_g, bn1_b, 128)
    x2 = _sc_gather_max(h1, adj2, so1, 128)
    h2, so2, ssum, scnt = _tc_dense(x2, d0w, dense0_b, bn2_g, bn2_b, mem32)
    smax = _sc_segment(h2, mem32, so2)
    return _tc_final(ssum, scnt, so2, smax, dense1_W, dense1_b)


# speedup vs baseline: 3.7499x; 1.0773x over previous
import functools

import jax
import jax.numpy as jnp
from jax import lax
from jax.experimental import pallas as pl
from jax.experimental.pallas import tpu as pltpu
from jax.experimental.pallas import tpu_sc as plsc

N = 100000
PER = 10000
MAXD = 10
NG = 128
C = 80
NCH = 125
NW = 32

_ROT = [0] + [(29 - 3 * (10 - d)) % 32 for d in range(1, 11)]


def _mesh():
    return plsc.VectorSubcoreMesh(core_axis_name="c", subcore_axis_name="s")


def _wid():
    return lax.axis_index("s") * 2 + lax.axis_index("c")


def _prefetch_idx(adj_h, idx4, sidx, t0, nch):
    cps = [pltpu.async_copy(adj_h.at[t0 + 32 * kk], idx4.at[kk], sidx)
           for kk in range(3)]

    @pl.when(nch == 4)
    def _():
        pltpu.sync_copy(adj_h.at[t0 + 32 * 3], idx4.at[3])

    for cp in cps:
        cp.wait()


def _sc_gather_sum(src, adj2, feat):
    nls = feat // 16

    @functools.partial(
        pl.kernel,
        out_type=jax.ShapeDtypeStruct((N, feat), jnp.float32),
        mesh=_mesh(),
        scratch_types=[
            pltpu.VMEM((4, 16, C), jnp.int32),
            pltpu.VMEM((C, feat), jnp.float32),
            pltpu.VMEM((C, feat), jnp.float32),
            pltpu.VMEM((C, feat), jnp.float32),
            pltpu.VMEM((C, feat), jnp.float32),
            pltpu.VMEM((C, feat), jnp.float32),
            pltpu.SemaphoreType.DMA,
            pltpu.SemaphoreType.DMA,
            pltpu.SemaphoreType.DMA,
            pltpu.SemaphoreType.DMA,
            pltpu.SemaphoreType.DMA,
            pltpu.SemaphoreType.DMA,
        ],
    )
    def k(src_h, adj_h, out_h, idx4, cb0, cb1, cb2, ob0, ob1,
          sidx, cs0, cs1, cs2, os0, os1):
        w = _wid()
        cbs = [cb0, cb1, cb2]
        css = [cs0, cs1, cs2]

        def chunk(kk, ob, os_, d, start):
            c = start + 32 * kk
            node0 = (d - 1) * PER + c * C
            sl0 = lambda j: (j - 1) % 3
            pend = {0: pltpu.async_copy(src_h.at[idx4.at[kk, 0]], ob, os_)}
            for j in (1, 2):
                if j < d:
                    pend[j] = pltpu.async_copy(src_h.at[idx4.at[kk, j]],
                                               cbs[sl0(j)], css[sl0(j)])
            pend[0].wait()
            for j in range(1, d):
                pend[j].wait()
                if j + 2 < d:
                    pend[j + 2] = pltpu.async_copy(
                        src_h.at[idx4.at[kk, j + 2]],
                        cbs[sl0(j + 2)], css[sl0(j + 2)])

                @pl.loop(0, C)
                def _(i, j=j):
                    for cc in range(nls):
                        sl = pl.ds(cc * 16, 16)
                        ob[i, sl] = ob[i, sl] + cbs[sl0(j)][i, sl]

            return pltpu.async_copy(ob, out_h.at[pl.ds(node0, C)], os_)

        for d in range(1, MAXD + 1):
            start = lax.rem(w + _ROT[d], 32)
            nch = jnp.where(start < 29, 4, 3).astype(jnp.int32)
            t0 = (d - 1) * NCH + start
            _prefetch_idx(adj_h, idx4, sidx, t0, nch)

            @pl.loop(0, 2)
            def _(p, d=d, start=start, nch=nch):
                o_a = chunk(2 * p, ob0, os0, d, start)
                more = 2 * p + 1 < nch

                @pl.when(more)
                def _(p=p, d=d, start=start):
                    o_b = chunk(2 * p + 1, ob1, os1, d, start)
                    o_a.wait()
                    o_b.wait()

                @pl.when(jnp.logical_not(more))
                def _():
                    o_a.wait()

    return k(src, adj2)


def _sc_gather_max(src, adj2, so, feat):
    nls = feat // 16

    @functools.partial(
        pl.kernel,
        out_type=jax.ShapeDtypeStruct((N, feat), jnp.float32),
        mesh=_mesh(),
        scratch_types=[
            pltpu.VMEM((4, 16, C), jnp.int32),
            pltpu.VMEM((C, feat), jnp.float32),
            pltpu.VMEM((C, feat), jnp.float32),
            pltpu.VMEM((C, feat), jnp.float32),
            pltpu.VMEM((C, feat), jnp.float32),
            pltpu.VMEM((C, feat), jnp.float32),
            pltpu.VMEM((C, feat), jnp.float32),
            pltpu.VMEM((C, feat), jnp.float32),
            pltpu.VMEM((2, feat), jnp.float32),
            pltpu.SemaphoreType.DMA,
            pltpu.SemaphoreType.DMA,
            pltpu.SemaphoreType.DMA,
            pltpu.SemaphoreType.DMA,
            pltpu.SemaphoreType.DMA,
            pltpu.SemaphoreType.DMA,
            pltpu.SemaphoreType.DMA,
            pltpu.SemaphoreType.DMA,
        ],
    )
    def k(src_h, adj_h, so_h, out_h, idx4, cb0, cb1, cb2, sb0, sb1, ob0, ob1,
          so_v, sidx, cs0, cs1, cs2, ss0, ss1, os0, os1):
        w = _wid()
        cbs = [cb0, cb1, cb2]
        css = [cs0, cs1, cs2]
        pltpu.sync_copy(so_h, so_v)
        scs = [so_v[0, pl.ds(cc * 16, 16)] for cc in range(nls)]
        ofs = [so_v[1, pl.ds(cc * 16, 16)] for cc in range(nls)]

        def chunk(kk, sb, ss_, ob, os_, d, start):
            c = start + 32 * kk
            node0 = (d - 1) * PER + c * C
            sp = pltpu.async_copy(src_h.at[pl.ds(node0, C)], sb, ss_)
            pend = {}
            for j in (0, 1, 2):
                if j < d:
                    pend[j] = pltpu.async_copy(src_h.at[idx4.at[kk, j]],
                                               cbs[j % 3], css[j % 3])
            sp.wait()

            @pl.loop(0, C)
            def _(i):
                for cc in range(nls):
                    sl = pl.ds(cc * 16, 16)
                    ob[i, sl] = sb[i, sl] * scs[cc] + ofs[cc]

            for j in range(d):
                pend[j].wait()

                @pl.loop(0, C)
                def _(i, j=j):
                    for cc in range(nls):
                        sl = pl.ds(cc * 16, 16)
                        r = cbs[j % 3][i, sl] * scs[cc] + ofs[cc]
                        ob[i, sl] = jnp.maximum(ob[i, sl], r)

                if j + 3 < d:
                    pend[j + 3] = pltpu.async_copy(
                        src_h.at[idx4.at[kk, j + 3]], cbs[j % 3], css[j % 3])

            return pltpu.async_copy(ob, out_h.at[pl.ds(node0, C)], os_)

        for d in range(1, MAXD + 1):
            start = lax.rem(w + _ROT[d], 32)
            nch = jnp.where(start < 29, 4, 3).astype(jnp.int32)
            t0 = (d - 1) * NCH + start
            _prefetch_idx(adj_h, idx4, sidx, t0, nch)

            @pl.loop(0, 2)
            def _(p, d=d, start=start, nch=nch):
                o_a = chunk(2 * p, sb0, ss0, ob0, os0, d, start)
                more = 2 * p + 1 < nch

                @pl.when(more)
                def _(p=p, d=d, start=start):
                    o_b = chunk(2 * p + 1, sb1, ss1, ob1, os1, d, start)
                    o_a.wait()
                    o_b.wait()

                @pl.when(jnp.logical_not(more))
                def _():
                    o_a.wait()

    return k(src, adj2, so)


def _sc_segment(h2, mem, so):
    nls = 8

    @functools.partial(
        pl.kernel,
        out_type=jax.ShapeDtypeStruct((2, NG, 128), jnp.float32),
        mesh=_mesh(),
        scratch_types=[
            pltpu.VMEM((128,), jnp.int32),
            pltpu.VMEM((128, 128), jnp.float32),
            pltpu.VMEM((NG, 128), jnp.float32),
            pltpu.VMEM((2, 128), jnp.float32),
            pltpu.VMEM((8, 128), jnp.float32),
            pltpu.VMEM_SHARED((16, NG, 128), jnp.float32),
        ],
    )
    def k(h2_h, mem_h, so_h, max_o,
          memb0, hbuf0, max_t, so_v, redb, max_s):
        cid = lax.axis_index("c")
        sid = lax.axis_index("s")
        w = sid * 2 + cid
        pltpu.sync_copy(so_h, so_v)
        scs = [so_v[0, pl.ds(cc * 16, 16)] for cc in range(nls)]
        ofs = [so_v[1, pl.ds(cc * 16, 16)] for cc in range(nls)]
        neg = jnp.full((16,), -3.4e38, jnp.float32)

        @pl.loop(0, NG)
        def _(r):
            for cc in range(nls):
                max_t[r, pl.ds(cc * 16, 16)] = neg

        def group_body(i0, memb, hbuf):
            mvec = memb[pl.ds(i0, 16)]
            for ln in range(16):
                m = mvec[ln]
                i = i0 + ln
                for cc in range(nls):
                    sl = pl.ds(cc * 16, 16)
                    r = hbuf[i, sl] * scs[cc] + ofs[cc]
                    max_t[m, sl] = jnp.maximum(max_t[m, sl], r)

        nc = jnp.where(w < 13, 25, 24).astype(jnp.int32)

        @pl.loop(0, 25)
        def _(kk):
            @pl.when(kk < nc)
            def _(kk=kk):
                row0 = (w + 32 * kk) * 128
                pltpu.sync_copy(mem_h.at[pl.ds(row0, 128)], memb0)
                pltpu.sync_copy(h2_h.at[pl.ds(row0, 128)], hbuf0)

                @pl.loop(0, 128, step=16)
                def _(i0):
                    group_body(i0, memb0, hbuf0)

        @pl.when(w == 0)
        def _():
            pltpu.sync_copy(mem_h.at[pl.ds(99968, 32)],
                            memb0.at[pl.ds(0, 32)])
            pltpu.sync_copy(h2_h.at[pl.ds(99968, 32)],
                            hbuf0.at[pl.ds(0, 32)])

            @pl.loop(0, 32, step=16)
            def _(i0):
                group_body(i0, memb0, hbuf0)

        pltpu.sync_copy(max_t, max_s.at[sid])
        plsc.subcore_barrier()

        r0 = sid * 8
        for rr in range(8):
            for cc in range(nls):
                max_t[rr, pl.ds(cc * 16, 16)] = neg

        @pl.loop(0, 16)
        def _(t):
            pltpu.sync_copy(max_s.at[t, pl.ds(r0, 8), :], redb)
            for rr in range(8):
                for cc in range(nls):
                    sl = pl.ds(cc * 16, 16)
                    max_t[rr, sl] = jnp.maximum(max_t[rr, sl], redb[rr, sl])

        pltpu.sync_copy(max_t.at[pl.ds(0, 8)], max_o.at[cid, pl.ds(r0, 8), :])

    return k(h2, mem, so)


def _tc_gconv(neigh, selfx, wn, ws, b, g, bb, f_in):
    def body(n_ref, s_ref, wn_ref, ws_ref, b_ref, g_ref, bb_ref,
             h_ref, so_ref, acc_ref):
        d = pl.program_id(0)
        i = pl.program_id(1)

        @pl.when((d == 0) & (i == 0))
        def _():
            acc_ref[...] = jnp.zeros_like(acc_ref)

        h = jnp.dot(n_ref[...], wn_ref[0], preferred_element_type=jnp.float32)
        h = h + jnp.dot(s_ref[...], ws_ref[0],
                        preferred_element_type=jnp.float32)
        h = jnp.maximum(h + b_ref[0, 0], 0.0)
        h_ref[...] = jnp.concatenate(
            [h, jnp.zeros((1000, 64), jnp.float32)], axis=1)
        acc_ref[0, :] = acc_ref[0, :] + jnp.sum(h, axis=0)
        acc_ref[1, :] = acc_ref[1, :] + jnp.sum(h * h, axis=0)

        @pl.when((d == 9) & (i == 9))
        def _():
            mean = acc_ref[0, :] / N
            var = acc_ref[1, :] / N - mean * mean
            scale = g_ref[0] * lax.rsqrt(var + 1e-5)
            pad = jnp.zeros((64,), jnp.float32)
            so_ref[0, :] = jnp.concatenate([scale, pad])
            so_ref[1, :] = jnp.concatenate([bb_ref[0] - mean * scale, pad])

    return pl.pallas_call(
        body,
        grid=(10, 10),
        in_specs=[
            pl.BlockSpec((1000, f_in), lambda d, i: (d * 10 + i, 0)),
            pl.BlockSpec((1000, f_in), lambda d, i: (d * 10 + i, 0)),
            pl.BlockSpec((1, f_in, 64), lambda d, i: (d, 0, 0)),
            pl.BlockSpec((1, f_in, 64), lambda d, i: (d, 0, 0)),
            pl.BlockSpec((1, 1, 64), lambda d, i: (d, 0, 0)),
            pl.BlockSpec((1, 64), lambda d, i: (0, 0)),
            pl.BlockSpec((1, 64), lambda d, i: (0, 0)),
        ],
        out_specs=[
            pl.BlockSpec((1000, 128), lambda d, i: (d * 10 + i, 0)),
            pl.BlockSpec((2, 128), lambda d, i: (0, 0)),
        ],
        out_shape=[
            jax.ShapeDtypeStruct((N, 128), jnp.float32),
            jax.ShapeDtypeStruct((2, 128), jnp.float32),
        ],
        scratch_shapes=[pltpu.VMEM((2, 64), jnp.float32)],
    )(neigh, selfx, wn, ws, b.reshape(10, 1, 64), g.reshape(1, -1),
      bb.reshape(1, -1))


def _tc_dense(x2, w, b, g, bb, mem):
    def body(x_ref, w_ref, b_ref, g_ref, bb_ref, m_ref,
             h_ref, so_ref, seg_ref, cnt_ref, acc_ref, sacc_ref, cacc_ref):
        i = pl.program_id(0)

        @pl.when(i == 0)
        def _():
            acc_ref[...] = jnp.zeros_like(acc_ref)
            sacc_ref[...] = jnp.zeros_like(sacc_ref)
            cacc_ref[...] = jnp.zeros_like(cacc_ref)

        h = jnp.dot(x_ref[...], w_ref[...], preferred_element_type=jnp.float32)
        h = jnp.maximum(h + b_ref[0], 0.0)
        h_ref[...] = h
        acc_ref[0, :] = acc_ref[0, :] + jnp.sum(h, axis=0)
        acc_ref[1, :] = acc_ref[1, :] + jnp.sum(h * h, axis=0)

        seg_ids = jax.lax.broadcasted_iota(jnp.int32, (1000, NG), 1)
        onehot = jnp.where(m_ref[0, 0][:, None] == seg_ids, 1.0, 0.0
                           ).astype(jnp.float32)
        sacc_ref[...] = sacc_ref[...] + jax.lax.dot_general(
            onehot, h, (((0,), (0,)), ((), ())),
            preferred_element_type=jnp.float32)
        cacc_ref[0, :] = cacc_ref[0, :] + jnp.sum(onehot, axis=0)

        @pl.when(i == 99)
        def _():
            mean = acc_ref[0, :] / N
            var = acc_ref[1, :] / N - mean * mean
            scale = g_ref[0] * lax.rsqrt(var + 1e-5)
            so_ref[0, :] = scale
            so_ref[1, :] = bb_ref[0] - mean * scale
            seg_ref[...] = sacc_ref[...]
            cnt_ref[...] = cacc_ref[...]

    return pl.pallas_call(
        body,
        grid=(100,),
        in_specs=[
            pl.BlockSpec((1000, 128), lambda i: (i, 0)),
            pl.BlockSpec((128, 128), lambda i: (0, 0)),
            pl.BlockSpec((1, 128), lambda i: (0, 0)),
            pl.BlockSpec((1, 128), lambda i: (0, 0)),
            pl.BlockSpec((1, 128), lambda i: (0, 0)),
            pl.BlockSpec((1, 1, 1000), lambda i: (i, 0, 0)),
        ],
        out_specs=[
            pl.BlockSpec((1000, 128), lambda i: (i, 0)),
            pl.BlockSpec((2, 128), lambda i: (0, 0)),
            pl.BlockSpec((NG, 128), lambda i: (0, 0)),
            pl.BlockSpec((1, NG), lambda i: (0, 0)),
        ],
        out_shape=[
            jax.ShapeDtypeStruct((N, 128), jnp.float32),
            jax.ShapeDtypeStruct((2, 128), jnp.float32),
            jax.ShapeDtypeStruct((NG, 128), jnp.float32),
            jax.ShapeDtypeStruct((1, NG), jnp.float32),
        ],
        scratch_shapes=[pltpu.VMEM((2, 128), jnp.float32),
                        pltpu.VMEM((NG, 128), jnp.float32),
                        pltpu.VMEM((1, NG), jnp.float32)],
    )(x2, w, b.reshape(1, -1), g.reshape(1, -1), bb.reshape(1, -1),
      mem.reshape(100, 1, 1000))


def _tc_final(ssum, scnt, so, smax, w1, b1):
    def body(s_ref, c_ref, so_ref, m_ref, w_ref, b_ref, o_ref):
        m = jnp.maximum(m_ref[0], m_ref[1])
        mean = s_ref[...] / c_ref[0][:, None]
        mean = mean * so_ref[0][None, :] + so_ref[1][None, :]
        gg = jnp.concatenate([mean, m], axis=1)
        o_ref[...] = jnp.dot(gg, w_ref[...],
                             preferred_element_type=jnp.float32) + b_ref[0]

    return pl.pallas_call(
        body,
        out_shape=jax.ShapeDtypeStruct((NG, 2), jnp.float32),
    )(ssum, scnt, so, smax, w1, b1.reshape(1, -1))


def kernel(node_features, deg_slice, membership, gc0_W, gc0_b, gc1_W, gc1_b,
           bn0_g, bn0_b, bn1_g, bn1_b, dense0_W, dense0_b, bn2_g, bn2_b,
           dense1_W, dense1_b, deg_adj_1, deg_adj_2, deg_adj_3, deg_adj_4,
           deg_adj_5, deg_adj_6, deg_adj_7, deg_adj_8, deg_adj_9, deg_adj_10):
    adjs = [deg_adj_1, deg_adj_2, deg_adj_3, deg_adj_4, deg_adj_5,
            deg_adj_6, deg_adj_7, deg_adj_8, deg_adj_9, deg_adj_10]
    parts = []
    for d, a in enumerate(adjs, 1):
        a32 = a.astype(jnp.int32)
        p = a32.reshape(NCH, C, d).transpose(0, 2, 1)
        p = jnp.pad(p, ((0, 0), (0, 16 - d), (0, 0)))
        parts.append(p)
    adj2 = jnp.concatenate(parts, axis=0)
    mem32 = membership.astype(jnp.int32)

    wn0, ws0 = gc0_W[1::2], gc0_W[2::2]
    b0 = gc0_b[1::2] + gc0_b[2::2]
    pad_w = ((0, 0), (0, 64), (0, 0))
    wn1 = jnp.pad(gc1_W[1::2], pad_w)
    ws1 = jnp.pad(gc1_W[2::2], pad_w)
    b1 = gc1_b[1::2] + gc1_b[2::2]
    d0w = jnp.pad(dense0_W, ((0, 64), (0, 0)))

    neigh0 = _sc_gather_sum(node_features, adj2, 128)
    h0, so0 = _tc_gconv(neigh0, node_features, wn0, ws0, b0, bn0_g, bn0_b, 128)
    x1 = _sc_gather_max(h0, adj2, so0, 128)
    neigh1 = _sc_gather_sum(x1, adj2, 128)
    h1, so1 = _tc_gconv(neigh1, x1, wn1, ws1, b1, bn1_g, bn1_b, 128)
    x2 = _sc_gather_max(h1, adj2, so1, 128)
    h2, so2, ssum, scnt = _tc_dense(x2, d0w, dense0_b, bn2_g, bn2_b, mem32)
    smax = _sc_segment(h2, mem32, so2)
    return _tc_final(ssum, scnt, so2, smax, dense1_W, dense1_b)
